# Initial kernel scaffold; baseline (speedup 1.0000x reference)
import jax
import jax.numpy as jnp
from jax import lax
from jax.experimental import pallas as pl
from jax.experimental.pallas import tpu as pltpu
from jax.experimental.pallas import tpu_sc as plsc

_CONF_THRES = 0.25
_IOU_THRES = 0.45
_MAX_ANCHORS = 50

_L = 16
_NPAD = 5008
_NCHUNK = _NPAD // _L
_OUTW = 16


def _nms_body(xt_hbm, out_hbm, buf, score, outv):
    wid = lax.axis_index("s") * 2 + lax.axis_index("c")

    @pl.when(wid < xt_hbm.shape[0])
    def _():
        b = wid
        pltpu.sync_copy(xt_hbm.at[b], buf)

        lane = lax.iota(jnp.int32, _L)
        ninf = jnp.full((_L,), -jnp.inf, jnp.float32)
        zeroi = jnp.zeros((_L,), jnp.int32)

        def argmax_fin(bv, bp):
            m = jnp.max(bv)
            bi = jnp.min(jnp.where(bv == m, bp, jnp.int32(2**30)))
            return m, bi

        def pre(ci, carry):
            bv, bp = carry
            off = ci * _L
            cf = buf[4, pl.ds(off, _L)]
            sc = jnp.where(cf > _CONF_THRES, cf, -jnp.inf)
            score[pl.ds(off, _L)] = sc
            upd = sc > bv
            return (jnp.where(upd, sc, bv),
                    jnp.where(upd, off + lane, bp))

        bv, bp = lax.fori_loop(0, _NCHUNK, pre, (ninf, zeroi))
        m0, bi0 = argmax_fin(bv, bp)

        def keep_step(k, carry):
            m, bi = carry
            valid = m > -jnp.inf
            iv = jnp.full((_L,), jnp.maximum(bi, 0), jnp.int32)
            r = jnp.zeros((_L,), jnp.int32)
            bx1 = plsc.load_gather(buf, [r, iv])
            by1 = plsc.load_gather(buf, [r + 1, iv])
            bx2 = plsc.load_gather(buf, [r + 2, iv])
            by2 = plsc.load_gather(buf, [r + 3, iv])
            clsv = plsc.load_gather(buf, [r + 5, iv])
            mv = jnp.full((_L,), m, jnp.float32)
            vals = jnp.where(lane == 0, bx1,
                   jnp.where(lane == 1, by1,
                   jnp.where(lane == 2, bx2,
                   jnp.where(lane == 3, by2,
                   jnp.where(lane == 4, mv,
                   jnp.where(lane == 5, clsv, jnp.float32(-1.0)))))))
            vals = jnp.where(valid, vals, jnp.float32(-1.0))
            outv[pl.ds(k * _L, _L)] = vals

            area_b = (jnp.maximum(bx2 - bx1, 0.0)
                      * jnp.maximum(by2 - by1, 0.0))
            biv = jnp.full((_L,), bi, jnp.int32)

            def sweep(ci, carry):
                bv, bp = carry
                off = ci * _L
                x1 = buf[0, pl.ds(off, _L)]
                y1 = buf[1, pl.ds(off, _L)]
                x2 = buf[2, pl.ds(off, _L)]
                y2 = buf[3, pl.ds(off, _L)]
                sc = score[pl.ds(off, _L)]
                ix1 = jnp.maximum(bx1, x1)
                iy1 = jnp.maximum(by1, y1)
                ix2 = jnp.minimum(bx2, x2)
                iy2 = jnp.minimum(by2, y2)
                inter = (jnp.maximum(ix2 - ix1, 0.0)
                         * jnp.maximum(iy2 - iy1, 0.0))
                areas = (jnp.maximum(x2 - x1, 0.0)
                         * jnp.maximum(y2 - y1, 0.0))
                union = area_b + areas - inter
                iou = inter / (union + 1e-9)
                gidx = off + lane
                supp = (iou >= _IOU_THRES) | (gidx == biv)
                sc = jnp.where(valid & supp, -jnp.inf, sc)
                score[pl.ds(off, _L)] = sc
                upd = sc > bv
                return (jnp.where(upd, sc, bv),
                        jnp.where(upd, gidx, bp))

            bv, bp = lax.fori_loop(0, _NCHUNK, sweep, (ninf, zeroi))
            return argmax_fin(bv, bp)

        lax.fori_loop(0, _MAX_ANCHORS, keep_step, (m0, bi0))
        pltpu.sync_copy(outv, out_hbm.at[b])


@jax.jit
def kernel(x):
    B, N, C = x.shape
    pad = jnp.zeros((B, _NPAD - N, C), x.dtype)
    pad = pad.at[:, :, 4].set(-1.0)
    xt = jnp.transpose(jnp.concatenate([x, pad], axis=1), (0, 2, 1))
    xt = jnp.asarray(xt, jnp.float32)

    run = pl.kernel(
        _nms_body,
        out_type=jax.ShapeDtypeStruct((B, _MAX_ANCHORS * _OUTW),
                                      jnp.float32),
        mesh=plsc.VectorSubcoreMesh(core_axis_name="c",
                                    subcore_axis_name="s"),
        scratch_types=[
            pltpu.VMEM((C, _NPAD), jnp.float32),
            pltpu.VMEM((_NPAD,), jnp.float32),
            pltpu.VMEM((_MAX_ANCHORS * _OUTW,), jnp.float32),
        ],
    )
    out = run(xt)
    return out.reshape(B, _MAX_ANCHORS, _OUTW)[:, :, :C]


# speedup vs baseline: 370.8893x; 370.8893x over previous
import jax
import jax.numpy as jnp
from jax import lax
from jax.experimental import pallas as pl
from jax.experimental.pallas import tpu as pltpu
from jax.experimental.pallas import tpu_sc as plsc

_CONF_THRES = 0.25
_IOU_THRES = 0.45
_MAX_ANCHORS = 50

_L = 16
_NPAD = 5008
_NCHUNK = _NPAD // _L
_OUTW = 16


def _nms_body(xt_hbm, out_hbm, buf, score, outv):
    wid = lax.axis_index("s") * 2 + lax.axis_index("c")

    @pl.when(wid < xt_hbm.shape[0])
    def _():
        b = wid
        pltpu.sync_copy(xt_hbm.at[b], buf)

        lane = lax.iota(jnp.int32, _L)
        ninf = jnp.full((_L,), -jnp.inf, jnp.float32)
        zeroi = jnp.zeros((_L,), jnp.int32)

        def argmax_fin(bv, bp):
            m = jnp.max(bv)
            bi = jnp.min(jnp.where(bv == m, bp, jnp.int32(2**30)))
            return m, bi

        def pre(ci, carry):
            bv, bp = carry
            off = ci * _L
            cf = buf[4, pl.ds(off, _L)]
            sc = jnp.where(cf > _CONF_THRES, cf, -jnp.inf)
            score[pl.ds(off, _L)] = sc
            upd = sc > bv
            return (jnp.where(upd, sc, bv),
                    jnp.where(upd, off + lane, bp))

        bv, bp = lax.fori_loop(0, _NCHUNK, pre, (ninf, zeroi))
        m0, bi0 = argmax_fin(bv, bp)

        def keep_step(k, carry):
            m, bi = carry
            valid = m > -jnp.inf
            iv = jnp.full((_L,), jnp.maximum(bi, 0), jnp.int32)
            r = jnp.zeros((_L,), jnp.int32)
            bx1 = plsc.load_gather(buf, [r, iv])
            by1 = plsc.load_gather(buf, [r + 1, iv])
            bx2 = plsc.load_gather(buf, [r + 2, iv])
            by2 = plsc.load_gather(buf, [r + 3, iv])
            clsv = plsc.load_gather(buf, [r + 5, iv])
            mv = jnp.full((_L,), m, jnp.float32)
            vals = jnp.where(lane == 0, bx1,
                   jnp.where(lane == 1, by1,
                   jnp.where(lane == 2, bx2,
                   jnp.where(lane == 3, by2,
                   jnp.where(lane == 4, mv,
                   jnp.where(lane == 5, clsv, jnp.float32(-1.0)))))))
            vals = jnp.where(valid, vals, jnp.float32(-1.0))
            outv[pl.ds(k * _L, _L)] = vals

            area_b = (jnp.maximum(bx2 - bx1, 0.0)
                      * jnp.maximum(by2 - by1, 0.0))
            biv = jnp.full((_L,), bi, jnp.int32)

            def sweep(ci, carry):
                bv, bp = carry
                off = ci * _L
                x1 = buf[0, pl.ds(off, _L)]
                y1 = buf[1, pl.ds(off, _L)]
                x2 = buf[2, pl.ds(off, _L)]
                y2 = buf[3, pl.ds(off, _L)]
                sc = score[pl.ds(off, _L)]
                ix1 = jnp.maximum(bx1, x1)
                iy1 = jnp.maximum(by1, y1)
                ix2 = jnp.minimum(bx2, x2)
                iy2 = jnp.minimum(by2, y2)
                inter = (jnp.maximum(ix2 - ix1, 0.0)
                         * jnp.maximum(iy2 - iy1, 0.0))
                areas = (jnp.maximum(x2 - x1, 0.0)
                         * jnp.maximum(y2 - y1, 0.0))
                union = area_b + areas - inter
                iou = inter / (union + 1e-9)
                gidx = off + lane
                supp = (iou >= _IOU_THRES) | (gidx == biv)
                sc = jnp.where(valid & supp, -jnp.inf, sc)
                score[pl.ds(off, _L)] = sc
                upd = sc > bv
                return (jnp.where(upd, sc, bv),
                        jnp.where(upd, gidx, bp))

            bv, bp = lax.fori_loop(0, _NCHUNK, sweep, (ninf, zeroi))
            return argmax_fin(bv, bp)

        lax.fori_loop(0, _MAX_ANCHORS, keep_step, (m0, bi0))
        pltpu.sync_copy(outv, out_hbm.at[b])


@jax.jit
def kernel(x):
    B, N, C = x.shape
    pad = jnp.zeros((B, _NPAD - N, C), x.dtype)
    pad = pad.at[:, :, 4].set(-1.0)
    xt = jnp.transpose(jnp.concatenate([x, pad], axis=1), (0, 2, 1))
    xt = jnp.asarray(xt, jnp.float32)

    run = pl.kernel(
        _nms_body,
        out_type=jax.ShapeDtypeStruct((B, _MAX_ANCHORS * _OUTW),
                                      jnp.float32),
        mesh=plsc.VectorSubcoreMesh(core_axis_name="c",
                                    subcore_axis_name="s"),
        scratch_types=[
            pltpu.VMEM((C, _NPAD), jnp.float32),
            pltpu.VMEM((_NPAD,), jnp.float32),
            pltpu.VMEM((_MAX_ANCHORS * _OUTW,), jnp.float32),
        ],
        compiler_params=pltpu.CompilerParams(needs_layout_passes=False),
    )
    out = run(xt)
    return out.reshape(B, _MAX_ANCHORS, _OUTW)[:, :, :C]


# speedup vs baseline: 385.9330x; 1.0406x over previous
import jax
import jax.numpy as jnp
from jax import lax
from jax.experimental import pallas as pl
from jax.experimental.pallas import tpu as pltpu
from jax.experimental.pallas import tpu_sc as plsc

_CONF_THRES = 0.25
_IOU_THRES = 0.45
_MAX_ANCHORS = 50

_L = 16
_UNROLL = 4
_NPAD = 5120
_NCHUNK = _NPAD // _L
_NITER = _NCHUNK // _UNROLL
_OUTW = 16


def _nms_body(xt_hbm, out_hbm, buf, score, area, outv):
    wid = lax.axis_index("s") * 2 + lax.axis_index("c")

    @pl.when(wid < xt_hbm.shape[0])
    def _():
        b = wid
        pltpu.sync_copy(xt_hbm.at[b], buf)

        lane = lax.iota(jnp.int32, _L)
        ninf = jnp.full((_L,), -jnp.inf, jnp.float32)
        zeroi = jnp.zeros((_L,), jnp.int32)

        def argmax_fin(accs):
            bv, bp = accs[0]
            for ov, op_ in accs[1:]:
                take = (ov > bv) | ((ov == bv) & (op_ < bp))
                bv = jnp.where(take, ov, bv)
                bp = jnp.where(take, op_, bp)
            m = jnp.max(bv)
            bi = jnp.min(jnp.where(bv == m, bp, jnp.int32(2**30)))
            return m, bi

        def pre(ci, carry):
            accs = list(carry)
            for u in range(_UNROLL):
                off = ci * (_L * _UNROLL) + u * _L
                sl = pl.ds(off, _L)
                x1 = buf[0, sl]
                y1 = buf[1, sl]
                x2 = buf[2, sl]
                y2 = buf[3, sl]
                cf = buf[4, sl]
                area[sl] = (jnp.maximum(x2 - x1, 0.0)
                            * jnp.maximum(y2 - y1, 0.0))
                sc = jnp.where(cf > _CONF_THRES, cf, -jnp.inf)
                score[sl] = sc
                bv, bp = accs[u]
                upd = sc > bv
                accs[u] = (jnp.where(upd, sc, bv),
                           jnp.where(upd, off + lane, bp))
            return tuple(accs)

        init = tuple((ninf, zeroi) for _ in range(_UNROLL))
        accs = lax.fori_loop(0, _NITER, pre, init)
        m0, bi0 = argmax_fin(accs)

        def emit(k, m, bi):
            valid = m > -jnp.inf
            iv = jnp.full((_L,), jnp.maximum(bi, 0), jnp.int32)
            r = jnp.zeros((_L,), jnp.int32)
            bx1 = plsc.load_gather(buf, [r, iv])
            by1 = plsc.load_gather(buf, [r + 1, iv])
            bx2 = plsc.load_gather(buf, [r + 2, iv])
            by2 = plsc.load_gather(buf, [r + 3, iv])
            clsv = plsc.load_gather(buf, [r + 5, iv])
            mv = jnp.full((_L,), m, jnp.float32)
            vals = jnp.where(lane == 0, bx1,
                   jnp.where(lane == 1, by1,
                   jnp.where(lane == 2, bx2,
                   jnp.where(lane == 3, by2,
                   jnp.where(lane == 4, mv,
                   jnp.where(lane == 5, clsv, jnp.float32(-1.0)))))))
            vals = jnp.where(valid, vals, jnp.float32(-1.0))
            outv[pl.ds(k * _L, _L)] = vals
            bx1 = jnp.where(valid, bx1, jnp.float32(0.0))
            by1 = jnp.where(valid, by1, jnp.float32(0.0))
            bx2 = jnp.where(valid, bx2, jnp.float32(-1.0))
            by2 = jnp.where(valid, by2, jnp.float32(-1.0))
            biv = jnp.where(valid, jnp.full((_L,), bi, jnp.int32), -1)
            return bx1, by1, bx2, by2, biv

        def keep_step(k, carry):
            m, bi = carry
            bx1, by1, bx2, by2, biv = emit(k, m, bi)

            area_b = (jnp.maximum(bx2 - bx1, 0.0)
                      * jnp.maximum(by2 - by1, 0.0))

            def sweep(ci, carry):
                accs = list(carry)
                for u in range(_UNROLL):
                    off = ci * (_L * _UNROLL) + u * _L
                    sl = pl.ds(off, _L)
                    x1 = buf[0, sl]
                    y1 = buf[1, sl]
                    x2 = buf[2, sl]
                    y2 = buf[3, sl]
                    sc = score[sl]
                    ar = area[sl]
                    ix1 = jnp.maximum(bx1, x1)
                    iy1 = jnp.maximum(by1, y1)
                    ix2 = jnp.minimum(bx2, x2)
                    iy2 = jnp.minimum(by2, y2)
                    inter = (jnp.maximum(ix2 - ix1, 0.0)
                             * jnp.maximum(iy2 - iy1, 0.0))
                    union = area_b + ar - inter
                    iou = inter / (union + 1e-9)
                    gidx = off + lane
                    supp = (iou >= _IOU_THRES) | (gidx == biv)
                    sc = jnp.where(supp, -jnp.inf, sc)
                    score[sl] = sc
                    bv, bp = accs[u]
                    upd = sc > bv
                    accs[u] = (jnp.where(upd, sc, bv),
                               jnp.where(upd, gidx, bp))
                return tuple(accs)

            accs = lax.fori_loop(0, _NITER, sweep, init)
            return argmax_fin(accs)

        m, bi = lax.fori_loop(0, _MAX_ANCHORS - 1, keep_step, (m0, bi0))
        emit(_MAX_ANCHORS - 1, m, bi)
        pltpu.sync_copy(outv, out_hbm.at[b])


@jax.jit
def kernel(x):
    B, N, C = x.shape
    pad = jnp.zeros((B, _NPAD - N, C), x.dtype)
    pad = pad.at[:, :, 4].set(-1.0)
    xt = jnp.transpose(jnp.concatenate([x, pad], axis=1), (0, 2, 1))
    xt = jnp.asarray(xt, jnp.float32)

    run = pl.kernel(
        _nms_body,
        out_type=jax.ShapeDtypeStruct((B, _MAX_ANCHORS * _OUTW),
                                      jnp.float32),
        mesh=plsc.VectorSubcoreMesh(core_axis_name="c",
                                    subcore_axis_name="s"),
        scratch_types=[
            pltpu.VMEM((C, _NPAD), jnp.float32),
            pltpu.VMEM((_NPAD,), jnp.float32),
            pltpu.VMEM((_NPAD,), jnp.float32),
            pltpu.VMEM((_MAX_ANCHORS * _OUTW,), jnp.float32),
        ],
        compiler_params=pltpu.CompilerParams(needs_layout_passes=False),
    )
    out = run(xt)
    return out.reshape(B, _MAX_ANCHORS, _OUTW)[:, :, :C]


# speedup vs baseline: 1193.2976x; 3.0920x over previous
import jax
import jax.numpy as jnp
from jax import lax
from jax.experimental import pallas as pl
from jax.experimental.pallas import tpu as pltpu
from jax.experimental.pallas import tpu_sc as plsc

_CONF_THRES = 0.25
_IOU_THRES = 0.45
_MAX_ANCHORS = 50

_L = 16
_UNROLL = 4
_NPAD = 5120
_OUTW = 16


def _nms_body(xt_hbm, out_hbm, buf, score, area, outv):
    wid = lax.axis_index("s") * 2 + lax.axis_index("c")

    @pl.when(wid < xt_hbm.shape[0])
    def _():
        b = wid
        pltpu.sync_copy(xt_hbm.at[b], buf)

        lane = lax.iota(jnp.int32, _L)
        ninf = jnp.full((_L,), -jnp.inf, jnp.float32)
        init = (ninf, jnp.full((_L,), jnp.int32(2**30)))

        def acc_update(bv, bp, sc, gidx):
            upd = (sc > bv) | ((sc == bv) & (gidx < bp))
            return jnp.where(upd, sc, bv), jnp.where(upd, gidx, bp)

        def argmax_fin(bv, bp):
            m = jnp.max(bv)
            bi = jnp.min(jnp.where(bv == m, bp, jnp.int32(2**30)))
            return m, bi

        @plsc.parallel_loop(0, _NPAD, _L, unroll=_UNROLL, carry=init)
        def pre_acc(off, carry):
            bv, bp = carry
            sl = pl.ds(off, _L)
            x1 = buf[0, sl]
            y1 = buf[1, sl]
            x2 = buf[2, sl]
            y2 = buf[3, sl]
            cf = buf[4, sl]
            area[sl] = (jnp.maximum(x2 - x1, 0.0)
                        * jnp.maximum(y2 - y1, 0.0))
            sc = jnp.where(cf > _CONF_THRES, cf, -jnp.inf)
            score[sl] = sc
            return acc_update(bv, bp, sc, off + lane)

        m0, bi0 = argmax_fin(*pre_acc)

        def emit(k, m, bi):
            valid = m > -jnp.inf
            iv = jnp.full((_L,), jnp.maximum(bi, 0), jnp.int32)
            r = jnp.zeros((_L,), jnp.int32)
            bx1 = plsc.load_gather(buf, [r, iv])
            by1 = plsc.load_gather(buf, [r + 1, iv])
            bx2 = plsc.load_gather(buf, [r + 2, iv])
            by2 = plsc.load_gather(buf, [r + 3, iv])
            clsv = plsc.load_gather(buf, [r + 5, iv])
            mv = jnp.full((_L,), m, jnp.float32)
            vals = jnp.where(lane == 0, bx1,
                   jnp.where(lane == 1, by1,
                   jnp.where(lane == 2, bx2,
                   jnp.where(lane == 3, by2,
                   jnp.where(lane == 4, mv,
                   jnp.where(lane == 5, clsv, jnp.float32(-1.0)))))))
            vals = jnp.where(valid, vals, jnp.float32(-1.0))
            outv[pl.ds(k * _L, _L)] = vals
            bx1 = jnp.where(valid, bx1, jnp.float32(0.0))
            by1 = jnp.where(valid, by1, jnp.float32(0.0))
            bx2 = jnp.where(valid, bx2, jnp.float32(-1.0))
            by2 = jnp.where(valid, by2, jnp.float32(-1.0))
            biv = jnp.where(valid, jnp.full((_L,), bi, jnp.int32), -1)
            return bx1, by1, bx2, by2, biv

        def keep_step(k, carry):
            m, bi = carry
            bx1, by1, bx2, by2, biv = emit(k, m, bi)
            area_b = (jnp.maximum(bx2 - bx1, 0.0)
                      * jnp.maximum(by2 - by1, 0.0))

            @plsc.parallel_loop(0, _NPAD, _L, unroll=_UNROLL, carry=init)
            def acc(off, carry):
                bv, bp = carry
                sl = pl.ds(off, _L)
                x1 = buf[0, sl]
                y1 = buf[1, sl]
                x2 = buf[2, sl]
                y2 = buf[3, sl]
                sc = score[sl]
                ar = area[sl]
                ix1 = jnp.maximum(bx1, x1)
                iy1 = jnp.maximum(by1, y1)
                ix2 = jnp.minimum(bx2, x2)
                iy2 = jnp.minimum(by2, y2)
                inter = (jnp.maximum(ix2 - ix1, 0.0)
                         * jnp.maximum(iy2 - iy1, 0.0))
                union = area_b + ar - inter
                iou = inter / (union + 1e-9)
                gidx = off + lane
                supp = (iou >= _IOU_THRES) | (gidx == biv)
                sc = jnp.where(supp, -jnp.inf, sc)
                score[sl] = sc
                return acc_update(bv, bp, sc, gidx)

            return argmax_fin(*acc)

        m, bi = lax.fori_loop(0, _MAX_ANCHORS - 1, keep_step, (m0, bi0))
        emit(_MAX_ANCHORS - 1, m, bi)
        pltpu.sync_copy(outv, out_hbm.at[b])


@jax.jit
def kernel(x):
    B, N, C = x.shape
    pad = jnp.zeros((B, _NPAD - N, C), x.dtype)
    pad = pad.at[:, :, 4].set(-1.0)
    xt = jnp.transpose(jnp.concatenate([x, pad], axis=1), (0, 2, 1))
    xt = jnp.asarray(xt, jnp.float32)

    run = pl.kernel(
        _nms_body,
        out_type=jax.ShapeDtypeStruct((B, _MAX_ANCHORS * _OUTW),
                                      jnp.float32),
        mesh=plsc.VectorSubcoreMesh(core_axis_name="c",
                                    subcore_axis_name="s"),
        scratch_types=[
            pltpu.VMEM((C, _NPAD), jnp.float32),
            pltpu.VMEM((_NPAD,), jnp.float32),
            pltpu.VMEM((_NPAD,), jnp.float32),
            pltpu.VMEM((_MAX_ANCHORS * _OUTW,), jnp.float32),
        ],
        compiler_params=pltpu.CompilerParams(needs_layout_passes=False),
    )
    out = run(xt)
    return out.reshape(B, _MAX_ANCHORS, _OUTW)[:, :, :C]


# speedup vs baseline: 1294.7207x; 1.0850x over previous
import jax
import jax.numpy as jnp
from jax import lax
from jax.experimental import pallas as pl
from jax.experimental.pallas import tpu as pltpu
from jax.experimental.pallas import tpu_sc as plsc

_CONF_THRES = 0.25
_IOU_THRES = 0.45
_MAX_ANCHORS = 50

_L = 16
_UNROLL = 4
_NPAD = 5120
_OUTW = 16


def _nms_body(xt_hbm, out_hbm, buf, score, area, outv):
    wid = lax.axis_index("s") * 2 + lax.axis_index("c")

    @pl.when(wid < xt_hbm.shape[0])
    def _():
        b = wid
        pltpu.sync_copy(xt_hbm.at[b], buf)

        lane = lax.iota(jnp.int32, _L)
        ninf = jnp.full((_L,), -jnp.inf, jnp.float32)
        init = (ninf, jnp.full((_L,), jnp.int32(2**30)))

        def acc_update(bv, bp, sc, gidx):
            upd = (sc > bv) | ((sc == bv) & (gidx < bp))
            return jnp.where(upd, sc, bv), jnp.where(upd, gidx, bp)

        def argmax_fin(bv, bp):
            m = jnp.max(bv)
            bi = jnp.min(jnp.where(bv == m, bp, jnp.int32(2**30)))
            return m, bi

        @plsc.parallel_loop(0, _NPAD, _L, unroll=_UNROLL, carry=init)
        def pre_acc(off, carry):
            bv, bp = carry
            sl = pl.ds(off, _L)
            x1 = buf[0, sl]
            y1 = buf[1, sl]
            x2 = buf[2, sl]
            y2 = buf[3, sl]
            cf = buf[4, sl]
            area[sl] = (jnp.maximum(x2 - x1, 0.0)
                        * jnp.maximum(y2 - y1, 0.0))
            sc = jnp.where(cf > _CONF_THRES, cf, -jnp.inf)
            score[sl] = sc
            return acc_update(bv, bp, sc, off + lane)

        m0, bi0 = argmax_fin(*pre_acc)

        def emit(k, m, bi):
            valid = m > -jnp.inf
            iv = jnp.full((_L,), jnp.maximum(bi, 0), jnp.int32)
            r = jnp.zeros((_L,), jnp.int32)
            bx1 = plsc.load_gather(buf, [r, iv])
            by1 = plsc.load_gather(buf, [r + 1, iv])
            bx2 = plsc.load_gather(buf, [r + 2, iv])
            by2 = plsc.load_gather(buf, [r + 3, iv])
            clsv = plsc.load_gather(buf, [r + 5, iv])
            mv = jnp.full((_L,), m, jnp.float32)
            vals = jnp.where(lane == 0, bx1,
                   jnp.where(lane == 1, by1,
                   jnp.where(lane == 2, bx2,
                   jnp.where(lane == 3, by2,
                   jnp.where(lane == 4, mv,
                   jnp.where(lane == 5, clsv, jnp.float32(-1.0)))))))
            vals = jnp.where(valid, vals, jnp.float32(-1.0))
            outv[pl.ds(k * _L, _L)] = vals
            koff = (jnp.minimum(jnp.maximum(bi, 0), _NPAD - 1) // _L) * _L
            ksl = pl.ds(koff, _L)
            score[ksl] = jnp.where(koff + lane == bi, -jnp.inf, score[ksl])
            bx1 = jnp.where(valid, bx1, jnp.float32(0.0))
            by1 = jnp.where(valid, by1, jnp.float32(0.0))
            bx2 = jnp.where(valid, bx2, jnp.float32(-1.0))
            by2 = jnp.where(valid, by2, jnp.float32(-1.0))
            return bx1, by1, bx2, by2

        def keep_step(k, carry):
            m, bi = carry
            bx1, by1, bx2, by2 = emit(k, m, bi)
            area_b = (jnp.maximum(bx2 - bx1, 0.0)
                      * jnp.maximum(by2 - by1, 0.0))

            @plsc.parallel_loop(0, _NPAD, _L, unroll=_UNROLL, carry=init)
            def acc(off, carry):
                bv, bp = carry
                sl = pl.ds(off, _L)
                x1 = buf[0, sl]
                y1 = buf[1, sl]
                x2 = buf[2, sl]
                y2 = buf[3, sl]
                sc = score[sl]
                ar = area[sl]
                ix1 = jnp.maximum(bx1, x1)
                iy1 = jnp.maximum(by1, y1)
                ix2 = jnp.minimum(bx2, x2)
                iy2 = jnp.minimum(by2, y2)
                inter = (jnp.maximum(ix2 - ix1, 0.0)
                         * jnp.maximum(iy2 - iy1, 0.0))
                union = area_b + ar - inter
                iou = inter / (union + 1e-9)
                gidx = off + lane
                sc = jnp.where(iou >= _IOU_THRES, -jnp.inf, sc)
                score[sl] = sc
                return acc_update(bv, bp, sc, gidx)

            return argmax_fin(*acc)

        m, bi = lax.fori_loop(0, _MAX_ANCHORS - 1, keep_step, (m0, bi0))
        emit(_MAX_ANCHORS - 1, m, bi)
        pltpu.sync_copy(outv, out_hbm.at[b])


@jax.jit
def kernel(x):
    B, N, C = x.shape
    pad = jnp.zeros((B, _NPAD - N, C), x.dtype)
    pad = pad.at[:, :, 4].set(-1.0)
    xt = jnp.transpose(jnp.concatenate([x, pad], axis=1), (0, 2, 1))
    xt = jnp.asarray(xt, jnp.float32)

    run = pl.kernel(
        _nms_body,
        out_type=jax.ShapeDtypeStruct((B, _MAX_ANCHORS * _OUTW),
                                      jnp.float32),
        mesh=plsc.VectorSubcoreMesh(core_axis_name="c",
                                    subcore_axis_name="s"),
        scratch_types=[
            pltpu.VMEM((C, _NPAD), jnp.float32),
            pltpu.VMEM((_NPAD,), jnp.float32),
            pltpu.VMEM((_NPAD,), jnp.float32),
            pltpu.VMEM((_MAX_ANCHORS * _OUTW,), jnp.float32),
        ],
        compiler_params=pltpu.CompilerParams(needs_layout_passes=False),
    )
    out = run(xt)
    return out.reshape(B, _MAX_ANCHORS, _OUTW)[:, :, :C]
